# Initial kernel scaffold; baseline (speedup 1.0000x reference)
import jax
import jax.numpy as jnp
from jax.experimental import pallas as pl


def kernel(inputs, categories, mask_positions, category_embedding):
    raise NotImplementedError("write your pallas kernel here")


# speedup vs baseline: 2.0435x; 2.0435x over previous
import functools

import jax
import jax.numpy as jnp
from jax import lax
from jax.experimental import pallas as pl
from jax.experimental.pallas import tpu as pltpu
from jax.experimental.pallas import tpu_sc as plsc

B = 1024
S = 200
D = 128
N = B * S

NUM_CORES = 2
NUM_SUBCORES = 16
LANES = 16
NW = NUM_CORES * NUM_SUBCORES

TOK_PER_W = N // NW
T = 128
CHUNKS = TOK_PER_W // T


def _sc_body(in_hbm, cat_hbm, mask_hbm, table_hbm, out_hbm,
             idx_v, mask_v, in_v, emb_v, in_sem, gat_sem):
    wid = lax.axis_index("s") * NUM_CORES + lax.axis_index("c")

    def chunk_body(ci, carry):
        base = wid * TOK_PER_W + ci * T
        pltpu.sync_copy(cat_hbm.at[pl.ds(base, T)], idx_v)
        pltpu.sync_copy(mask_hbm.at[pl.ds(base, T)], mask_v)
        in_cp = pltpu.async_copy(in_hbm.at[pl.ds(base, T)], in_v, in_sem)
        gat_cp = pltpu.async_copy(table_hbm.at[idx_v], emb_v, gat_sem)
        in_cp.wait()
        gat_cp.wait()

        def grp_body(g, c):
            m16 = mask_v[pl.ds(g * LANES, LANES)]
            for j in range(LANES):
                keep = m16[j] == 0
                t = g * LANES + j
                for d in range(D // LANES):
                    x = in_v[t, pl.ds(d * LANES, LANES)]
                    e = emb_v[t, pl.ds(d * LANES, LANES)]
                    in_v[t, pl.ds(d * LANES, LANES)] = x * jnp.where(
                        keep, e, jnp.float32(1.0))
            return c

        lax.fori_loop(0, T // LANES, grp_body, 0)
        pltpu.sync_copy(in_v, out_hbm.at[pl.ds(base, T)])
        return carry

    lax.fori_loop(0, CHUNKS, chunk_body, 0)


@jax.jit
def _run(in_flat, cats, mask, table):
    mesh = plsc.VectorSubcoreMesh(
        core_axis_name="c", subcore_axis_name="s",
        num_cores=NUM_CORES, num_subcores=NUM_SUBCORES)
    fn = pl.kernel(
        _sc_body,
        out_type=jax.ShapeDtypeStruct((N, D), jnp.float32),
        mesh=mesh,
        scratch_types=[
            pltpu.VMEM((T,), jnp.int32),
            pltpu.VMEM((T,), jnp.int32),
            pltpu.VMEM((T, D), jnp.float32),
            pltpu.VMEM((T, D), jnp.float32),
            pltpu.SemaphoreType.DMA,
            pltpu.SemaphoreType.DMA,
        ],
    )
    return fn(in_flat, cats, mask, table)


def kernel(inputs, categories, mask_positions, category_embedding):
    in_flat = inputs.reshape(N, D)
    cats = categories.reshape(N).astype(jnp.int32)
    mask = mask_positions.reshape(N).astype(jnp.int32)
    out = _run(in_flat, cats, mask, category_embedding)
    return out.reshape(B, S, D)


# speedup vs baseline: 3.1674x; 1.5500x over previous
import jax
import jax.numpy as jnp
from jax import lax
from jax.experimental import pallas as pl
from jax.experimental.pallas import tpu as pltpu
from jax.experimental.pallas import tpu_sc as plsc

B = 1024
S = 200
D = 128
N = B * S

NUM_CORES = 2
NUM_SUBCORES = 16
LANES = 16
NW = NUM_CORES * NUM_SUBCORES

TOK_PER_W = N // NW
T = 128
CHUNKS = TOK_PER_W // T
NBUF = 2


def _sc_body(in_hbm, cat_hbm, mask_hbm, table_hbm, out_hbm,
             idx_all, mask_all, in_v, emb_v, out_v,
             in_sem, gat_sem, out_sem):
    wid = lax.axis_index("s") * NUM_CORES + lax.axis_index("c")
    tok0 = wid * TOK_PER_W

    pltpu.sync_copy(cat_hbm.at[wid], idx_all)
    pltpu.sync_copy(mask_hbm.at[wid], mask_all)

    def start_fetch(ci, b):
        base = tok0 + ci * T
        pltpu.async_copy(in_hbm.at[pl.ds(base, T)], in_v[b], in_sem[b])
        pltpu.async_copy(table_hbm.at[idx_all.at[ci]], emb_v[b], gat_sem[b])

    def wait_fetch(ci, b):
        base = tok0 + ci * T
        pltpu.make_async_copy(
            in_hbm.at[pl.ds(base, T)], in_v[b], in_sem[b]).wait()
        pltpu.make_async_copy(
            table_hbm.at[idx_all.at[ci]], emb_v[b], gat_sem[b]).wait()

    def wait_out(ci, b):
        base = tok0 + ci * T
        pltpu.make_async_copy(
            out_v[b], out_hbm.at[pl.ds(base, T)], out_sem[b]).wait()

    start_fetch(0, 0)

    def pair_body(cp, carry):
        for b in range(NBUF):
            ci = cp * NBUF + b
            nb = (b + 1) % NBUF

            @pl.when(ci + 1 < CHUNKS)
            def _():
                start_fetch(ci + 1, nb)

            wait_fetch(ci, b)

            def grp_body(g, c):
                m16 = mask_all[ci, pl.ds(g * LANES, LANES)]
                for j in range(LANES):
                    keep = m16[j] == 0
                    t = g * LANES + j
                    for d in range(D // LANES):
                        x = in_v[b][t, pl.ds(d * LANES, LANES)]
                        e = emb_v[b][t, pl.ds(d * LANES, LANES)]
                        out_v[b][t, pl.ds(d * LANES, LANES)] = x * jnp.where(
                            keep, e, jnp.float32(1.0))
                return c

            lax.fori_loop(0, T // LANES, grp_body, 0)

            @pl.when(ci >= 1)
            def _():
                wait_out(ci - 1, nb)

            base = tok0 + ci * T
            pltpu.async_copy(out_v[b], out_hbm.at[pl.ds(base, T)], out_sem[b])
        return carry

    lax.fori_loop(0, CHUNKS // NBUF, pair_body, 0)
    wait_out(CHUNKS - 1, (CHUNKS - 1) % NBUF)


@jax.jit
def _run(in_flat, cats, mask, table):
    mesh = plsc.VectorSubcoreMesh(
        core_axis_name="c", subcore_axis_name="s",
        num_cores=NUM_CORES, num_subcores=NUM_SUBCORES)
    fn = pl.kernel(
        _sc_body,
        out_type=jax.ShapeDtypeStruct((N, D), jnp.float32),
        mesh=mesh,
        scratch_types=[
            pltpu.VMEM((CHUNKS, T), jnp.int32),
            pltpu.VMEM((CHUNKS, T), jnp.int32),
            [pltpu.VMEM((T, D), jnp.float32) for _ in range(NBUF)],
            [pltpu.VMEM((T, D), jnp.float32) for _ in range(NBUF)],
            [pltpu.VMEM((T, D), jnp.float32) for _ in range(NBUF)],
            [pltpu.SemaphoreType.DMA for _ in range(NBUF)],
            [pltpu.SemaphoreType.DMA for _ in range(NBUF)],
            [pltpu.SemaphoreType.DMA for _ in range(NBUF)],
        ],
    )
    return fn(in_flat, cats, mask, table)


def kernel(inputs, categories, mask_positions, category_embedding):
    in_flat = inputs.reshape(N, D)
    cats = categories.reshape(NW, CHUNKS, T).astype(jnp.int32)
    mask = mask_positions.reshape(NW, CHUNKS, T).astype(jnp.int32)
    out = _run(in_flat, cats, mask, category_embedding)
    return out.reshape(B, S, D)
